# Initial kernel scaffold; baseline (speedup 1.0000x reference)
import jax
import jax.numpy as jnp
from jax.experimental import pallas as pl


def kernel(x, edge_index, W_in, b_in, W_u, b_u, W_v, W1, b1, W2, b2):
    raise NotImplementedError("write your pallas kernel here")


# speedup vs baseline: 44.4302x; 44.4302x over previous
import functools

import jax
import jax.numpy as jnp
import numpy as np
from jax import lax
from jax.experimental import pallas as pl
from jax.experimental.pallas import tpu as pltpu
from jax.experimental.pallas import tpu_sc as plsc

N = 10000
E = 320000
DIM = 128
H = 8
HD = DIM // H
HID = DIM * 2

NC = 2
NS = 16
ACCW = DIM + 16
EPT = E // (NC * NS)
C = 80
NCH = EPT // C
NP = 10240
RPT = NP // NS

_BN = 1000
_GRID = N // _BN


def _pre_body(x_ref, win_ref, bin_ref, wu2_ref, bu2_ref, wv2_ref,
              h_ref, su_ref, sv_ref):
    h = jnp.dot(x_ref[...], win_ref[...],
                preferred_element_type=jnp.float32) + bin_ref[...]
    h_ref[...] = h
    su_ref[...] = jnp.dot(h, wu2_ref[...],
                          preferred_element_type=jnp.float32) + bu2_ref[...]
    sv_ref[...] = jnp.dot(h, wv2_ref[...],
                          preferred_element_type=jnp.float32)


def _post_body(a0_ref, a1_ref, p_ref, w1_ref, b1_ref, w2_ref, b2_ref,
               out_ref):
    a = a0_ref[...] + a1_ref[...]
    den = jnp.dot(a[:, DIM:ACCW], p_ref[...],
                  preferred_element_type=jnp.float32)
    den = jnp.where(den <= 0.0, 1.0, den)
    agg = a[:, :DIM] / den
    f = jnp.dot(agg, w1_ref[...], preferred_element_type=jnp.float32)
    f = f + b1_ref[...]
    f = 0.5 * f * (1.0 + lax.erf(f * np.float32(1.0 / np.sqrt(2.0))))
    out_ref[...] = jnp.dot(f, w2_ref[...],
                           preferred_element_type=jnp.float32) + b2_ref[...]


def _edge_body(su_hbm, sv_hbm, h_hbm, src_hbm, dst_hbm, zeros_hbm, out_hbm,
               src_idx, dst_idx, su_buf, sv_buf, hrows, msg, acc,
               sem0, sem1, sem2):
    c = lax.axis_index("c")
    s = lax.axis_index("s")
    tile_lo = s * RPT
    pltpu.sync_copy(zeros_hbm.at[pl.ds(tile_lo, RPT)],
                    acc.at[pl.ds(tile_lo, RPT)])
    plsc.subcore_barrier()

    base_t = (c * NS + s) * EPT

    def chunk_body(j, _):
        base = base_t + j * C
        pltpu.sync_copy(src_hbm.at[pl.ds(base, C)], src_idx)
        pltpu.sync_copy(dst_hbm.at[pl.ds(base, C)], dst_idx)
        ga = pltpu.async_copy(su_hbm.at[src_idx], su_buf, sem0)
        gb = pltpu.async_copy(sv_hbm.at[dst_idx], sv_buf, sem1)
        gc = pltpu.async_copy(h_hbm.at[src_idx], hrows, sem2)
        ga.wait()
        gb.wait()
        gc.wait()

        def edge_body(e, _):
            t = su_buf[e, :] + sv_buf[e, :]
            t = jnp.where(t >= 0.0, t, 0.2 * t)
            w = jnp.exp(t)
            msg[e, pl.ds(DIM, 16)] = w
            for k in range(HD // 2):
                msg[e, pl.ds(k * 16, 16)] = hrows[e, pl.ds(k * 16, 16)] * w
            return 0

        lax.fori_loop(0, C, edge_body, 0)
        pltpu.sync_copy(msg, acc.at[dst_idx], add=True)
        return 0

    lax.fori_loop(0, NCH, chunk_body, 0)
    plsc.subcore_barrier()
    pltpu.sync_copy(acc.at[pl.ds(tile_lo, RPT)],
                    out_hbm.at[c, pl.ds(tile_lo, RPT)])


@functools.cache
def _edge_kernel():
    return pl.kernel(
        _edge_body,
        out_type=jax.ShapeDtypeStruct((NC, NP, ACCW), jnp.float32),
        mesh=plsc.VectorSubcoreMesh(core_axis_name="c", subcore_axis_name="s",
                                    num_cores=NC, num_subcores=NS),
        scratch_types=[
        pltpu.VMEM((C,), jnp.int32),
        pltpu.VMEM((C,), jnp.int32),
        pltpu.VMEM((C, 16), jnp.float32),
        pltpu.VMEM((C, 16), jnp.float32),
        pltpu.VMEM((C, DIM), jnp.float32),
        pltpu.VMEM((C, ACCW), jnp.float32),
        pltpu.VMEM_SHARED((NP, ACCW), jnp.float32),
        pltpu.SemaphoreType.DMA,
        pltpu.SemaphoreType.DMA,
        pltpu.SemaphoreType.DMA,
        ],
        compiler_params=pltpu.CompilerParams(use_tc_tiling_on_sc=False),
    )

_P = np.zeros((16, DIM), dtype=np.float32)
for _h in range(H):
    _P[_h, np.arange(HD) * H + _h] = 0.5
    _P[_h + H, np.arange(HD) * H + _h] = 0.5


@jax.jit
def kernel(x, edge_index, W_in, b_in, W_u, b_u, W_v, W1, b1, W2, b2):
    wu2 = jnp.concatenate([W_u, W_u], axis=1)
    wv2 = jnp.concatenate([W_v, W_v], axis=1)
    bu2 = jnp.concatenate([b_u, b_u])[None, :]

    h, surep, svrep = pl.pallas_call(
        _pre_body,
        grid=(_GRID,),
        in_specs=[
            pl.BlockSpec((_BN, DIM), lambda i: (i, 0)),
            pl.BlockSpec((DIM, DIM), lambda i: (0, 0)),
            pl.BlockSpec((1, DIM), lambda i: (0, 0)),
            pl.BlockSpec((DIM, 16), lambda i: (0, 0)),
            pl.BlockSpec((1, 16), lambda i: (0, 0)),
            pl.BlockSpec((DIM, 16), lambda i: (0, 0)),
        ],
        out_specs=[
            pl.BlockSpec((_BN, DIM), lambda i: (i, 0)),
            pl.BlockSpec((_BN, 16), lambda i: (i, 0)),
            pl.BlockSpec((_BN, 16), lambda i: (i, 0)),
        ],
        out_shape=[
            jax.ShapeDtypeStruct((N, DIM), jnp.float32),
            jax.ShapeDtypeStruct((N, 16), jnp.float32),
            jax.ShapeDtypeStruct((N, 16), jnp.float32),
        ],
    )(x, W_in, b_in.reshape(1, DIM), wu2, bu2, wv2)

    ei = edge_index.astype(jnp.int32)
    zeros = jnp.zeros((NP, ACCW), jnp.float32)
    accs = _edge_kernel()(surep, svrep, h, ei[0], ei[1], zeros)[:, :N, :]

    out = pl.pallas_call(
        _post_body,
        grid=(_GRID,),
        in_specs=[
            pl.BlockSpec((_BN, ACCW), lambda i: (i, 0)),
            pl.BlockSpec((_BN, ACCW), lambda i: (i, 0)),
            pl.BlockSpec((16, DIM), lambda i: (0, 0)),
            pl.BlockSpec((DIM, HID), lambda i: (0, 0)),
            pl.BlockSpec((1, HID), lambda i: (0, 0)),
            pl.BlockSpec((HID, DIM), lambda i: (0, 0)),
            pl.BlockSpec((1, DIM), lambda i: (0, 0)),
        ],
        out_specs=pl.BlockSpec((_BN, DIM), lambda i: (i, 0)),
        out_shape=jax.ShapeDtypeStruct((N, DIM), jnp.float32),
    )(accs[0], accs[1], _P, W1, b1.reshape(1, HID), W2, b2.reshape(1, DIM))
    return out
